# baseline (speedup 1.0000x reference)
import functools

import jax
import jax.numpy as jnp
from jax import lax
from jax.experimental import pallas as pl
from jax.experimental.pallas import tpu as pltpu
from jax.experimental.pallas import tpu_sc as plsc

_N = 1 << 24
_OUT = _N >> 1
_B = 20
_NW = 32
_PER_W = _OUT // _NW
_W_PER_CHUNK = (1 << _B) // _PER_W

_mesh = plsc.VectorSubcoreMesh(core_axis_name="c", subcore_axis_name="s")


@functools.partial(
    pl.kernel,
    mesh=_mesh,
    out_type=jax.ShapeDtypeStruct((_OUT,), jnp.float32),
)
def _restrict(psi_hbm, out_hbm):
    wid = lax.axis_index("s") * 2 + lax.axis_index("c")
    out_base = wid * _PER_W
    in_base = (wid // _W_PER_CHUNK) * (1 << (_B + 1)) + (wid % _W_PER_CHUNK) * _PER_W
    pltpu.sync_copy(
        psi_hbm.at[pl.ds(in_base, _PER_W)],
        out_hbm.at[pl.ds(out_base, _PER_W)],
    )


def kernel(psi):
    return _restrict(psi)
